# baseline (speedup 1.0000x reference)
import functools

import jax
import jax.numpy as jnp
from jax.experimental import pallas as pl
from jax.experimental.pallas import tpu as pltpu

_NUM_CLASSES = 5
_LANES = 128
_GROUPS = 8
_GW = _LANES // _GROUPS
_ROWS_PER_BLOCK = 8192
_VMEM_LIMIT = 48 << 20


def _packed_ce_kernel(x_ref, w_ref, b_ref, yp_ref, out_ref,
                      acc_ref, wbd_ref, bbd_ref, smat_ref, emat_ref,
                      *, steps, inv_b, d):
    i = pl.program_id(0)
    cr = x_ref.shape[0] // _GROUPS

    @pl.when(i == 0)
    def _init():
        acc_ref[...] = jnp.zeros_like(acc_ref)
        w = w_ref[...]
        bb = b_ref[...]
        wbd_ref[0:d, :] = w
        for g in range(1, _GROUPS):
            wbd_ref[g * d:(g + 1) * d, :] = pltpu.roll(w, g * _GW, axis=1)
            bb = jnp.maximum(bb, pltpu.roll(b_ref[...], g * _GW, axis=1))
        bbd_ref[...] = bb
        r16 = jax.lax.broadcasted_iota(jnp.int32, (_LANES, _LANES), 0) // _GW
        c16 = jax.lax.broadcasted_iota(jnp.int32, (_LANES, _LANES), 1) // _GW
        smat_ref[...] = (r16 == c16).astype(jnp.float32)
        ge = jax.lax.broadcasted_iota(jnp.int32, (_GROUPS, _LANES), 0)
        ce = jax.lax.broadcasted_iota(jnp.int32, (_GROUPS, _LANES), 1) // _GW
        emat_ref[...] = (ge == ce).astype(jnp.float32)

    logits = bbd_ref[...] + jnp.zeros((cr, _LANES), jnp.float32)
    for g in range(_GROUPS):
        logits += jnp.dot(x_ref[g * cr:(g + 1) * cr, :],
                          wbd_ref[g * d:(g + 1) * d, :],
                          preferred_element_type=jnp.float32)

    e = jnp.exp(logits)
    s = jnp.dot(e, smat_ref[...], preferred_element_type=jnp.float32)
    yx = jnp.dot(yp_ref[...], emat_ref[...], preferred_element_type=jnp.float32)

    lane = (jax.lax.broadcasted_iota(jnp.int32, (1, _LANES), 1)
            % _GW).astype(jnp.float32)
    lse_part = jnp.where(lane == 0.0, jnp.log(s), 0.0)
    pick_part = jnp.where(lane == yx, logits, 0.0)
    t = lse_part - pick_part
    acc_ref[...] += jnp.sum(t.reshape(cr // 8, 8, _LANES), axis=0)

    @pl.when(i == steps - 1)
    def _finalize():
        out_ref[...] = jnp.broadcast_to(jnp.sum(acc_ref[...]) * inv_b,
                                        out_ref.shape)


def kernel(x, w_t_pad, b_pad, y):
    batch, d = x.shape
    if batch % _ROWS_PER_BLOCK:
        return _unpacked_kernel(x, w_t_pad, b_pad, y)
    steps = batch // _ROWS_PER_BLOCK
    cr = _ROWS_PER_BLOCK // _GROUPS
    yp = (y.reshape(steps, _GROUPS, cr).swapaxes(1, 2)
          .reshape(steps * cr, _GROUPS).astype(jnp.float32))

    body = functools.partial(_packed_ce_kernel, steps=steps,
                             inv_b=1.0 / batch, d=d)
    cost = pl.CostEstimate(
        flops=2 * batch * d * _GW + 8 * batch * _GW,
        transcendentals=batch * _GW + batch,
        bytes_accessed=batch * d * 4 + d * _LANES * 4 + batch * 4,
    )
    out = pl.pallas_call(
        body,
        out_shape=jax.ShapeDtypeStruct((8, _LANES), jnp.float32),
        grid=(steps,),
        in_specs=[
            pl.BlockSpec((_ROWS_PER_BLOCK, d), lambda i: (i, 0)),
            pl.BlockSpec((d, _LANES), lambda i: (0, 0)),
            pl.BlockSpec((1, _LANES), lambda i: (0, 0)),
            pl.BlockSpec((cr, _GROUPS), lambda i: (i, 0)),
        ],
        out_specs=pl.BlockSpec((8, _LANES), lambda i: (0, 0)),
        scratch_shapes=[
            pltpu.VMEM((8, _LANES), jnp.float32),
            pltpu.VMEM((_GROUPS * d, _LANES), jnp.float32),
            pltpu.VMEM((1, _LANES), jnp.float32),
            pltpu.VMEM((_LANES, _LANES), jnp.float32),
            pltpu.VMEM((_GROUPS, _LANES), jnp.float32),
        ],
        compiler_params=pltpu.CompilerParams(
            dimension_semantics=("arbitrary",),
            vmem_limit_bytes=_VMEM_LIMIT,
        ),
        cost_estimate=cost,
    )(x, w_t_pad, b_pad, yp)
    return out[0, 0]


def _unpacked_ce_kernel(x_ref, wt_ref, b_ref, y_ref, out_ref, acc_ref,
                        *, steps, inv_b, total_rows):
    i = pl.program_id(0)
    tm = x_ref.shape[0]

    @pl.when(i == 0)
    def _init():
        acc_ref[...] = jnp.zeros_like(acc_ref)

    logits = jnp.dot(x_ref[...], wt_ref[...],
                     preferred_element_type=jnp.float32)
    logits = logits + b_ref[...]
    m = jnp.max(logits, axis=-1, keepdims=True)
    lse = m + jnp.log(jnp.sum(jnp.exp(logits - m), axis=-1, keepdims=True))
    col = jax.lax.broadcasted_iota(jnp.int32, logits.shape, 1)
    picked = jnp.sum(jnp.where(col == y_ref[...], logits, 0.0),
                     axis=-1, keepdims=True)
    loss = lse - picked
    row = jax.lax.broadcasted_iota(jnp.int32, (tm, 1), 0) + i * tm
    acc_ref[...] += jnp.where(row < total_rows, loss, 0.0)

    @pl.when(i == steps - 1)
    def _finalize():
        out_ref[...] = jnp.broadcast_to(
            jnp.sum(acc_ref[...]) * inv_b, out_ref.shape)


def _unpacked_kernel(x, w_t_pad, b_pad, y):
    batch, d = x.shape
    tm = min(2048, max(8, -(-batch // 8) * 8))
    steps = pl.cdiv(batch, tm)
    y2 = y.reshape(batch, 1).astype(jnp.int32)
    body = functools.partial(_unpacked_ce_kernel, steps=steps,
                             inv_b=1.0 / batch, total_rows=batch)
    out = pl.pallas_call(
        body,
        out_shape=jax.ShapeDtypeStruct((8, _LANES), jnp.float32),
        grid=(steps,),
        in_specs=[
            pl.BlockSpec((tm, d), lambda i: (i, 0)),
            pl.BlockSpec((d, _LANES), lambda i: (0, 0)),
            pl.BlockSpec((1, _LANES), lambda i: (0, 0)),
            pl.BlockSpec((tm, 1), lambda i: (i, 0)),
        ],
        out_specs=pl.BlockSpec((8, _LANES), lambda i: (0, 0)),
        scratch_shapes=[pltpu.VMEM((tm, 1), jnp.float32)],
        compiler_params=pltpu.CompilerParams(
            dimension_semantics=("arbitrary",),
            vmem_limit_bytes=_VMEM_LIMIT,
        ),
    )(x, w_t_pad, b_pad, y2)
    return out[0, 0]


# speedup vs baseline: 1.0167x; 1.0167x over previous
import functools

import jax
import jax.numpy as jnp
from jax.experimental import pallas as pl
from jax.experimental.pallas import tpu as pltpu

_NUM_CLASSES = 5
_LANES = 128
_GROUPS = 8
_GW = _LANES // _GROUPS
_ROWS_PER_BLOCK = 4096
_VMEM_LIMIT = 40 << 20


def _packed_ce_kernel(x_ref, w_ref, b_ref, yp_ref, out_ref,
                      acc_ref, wbd_ref, bbd_ref, smat_ref, emat_ref,
                      *, steps, inv_b, d):
    i = pl.program_id(0)
    cr = x_ref.shape[0] // _GROUPS

    @pl.when(i == 0)
    def _init():
        acc_ref[...] = jnp.zeros_like(acc_ref)
        w = w_ref[...]
        bb = b_ref[...]
        wbd_ref[0:d, :] = w
        for g in range(1, _GROUPS):
            wbd_ref[g * d:(g + 1) * d, :] = pltpu.roll(w, g * _GW, axis=1)
            bb = jnp.maximum(bb, pltpu.roll(b_ref[...], g * _GW, axis=1))
        bbd_ref[...] = bb
        r16 = jax.lax.broadcasted_iota(jnp.int32, (_LANES, _LANES), 0) // _GW
        c16 = jax.lax.broadcasted_iota(jnp.int32, (_LANES, _LANES), 1) // _GW
        smat_ref[...] = (r16 == c16).astype(jnp.float32)
        ge = jax.lax.broadcasted_iota(jnp.int32, (_GROUPS, _LANES), 0)
        ce = jax.lax.broadcasted_iota(jnp.int32, (_GROUPS, _LANES), 1) // _GW
        emat_ref[...] = (ge == ce).astype(jnp.float32)

    logits = bbd_ref[...] + jnp.zeros((cr, _LANES), jnp.float32)
    for g in range(_GROUPS):
        logits += jnp.dot(x_ref[g * cr:(g + 1) * cr, :],
                          wbd_ref[g * d:(g + 1) * d, :],
                          preferred_element_type=jnp.float32)

    e = jnp.exp(logits)
    s = jnp.dot(e, smat_ref[...], preferred_element_type=jnp.float32)
    yx = jnp.dot(yp_ref[...], emat_ref[...], preferred_element_type=jnp.float32)

    lane = (jax.lax.broadcasted_iota(jnp.int32, (1, _LANES), 1)
            % _GW).astype(jnp.float32)
    lse_part = jnp.where(lane == 0.0, jnp.log(s), 0.0)
    pick_part = jnp.where(lane == yx, logits, 0.0)
    t = lse_part - pick_part
    acc_ref[...] += jnp.sum(t.reshape(cr // 8, 8, _LANES), axis=0)

    @pl.when(i == steps - 1)
    def _finalize():
        out_ref[...] = jnp.broadcast_to(jnp.sum(acc_ref[...]) * inv_b,
                                        out_ref.shape)


def kernel(x, w_t_pad, b_pad, y):
    batch, d = x.shape
    if batch % _ROWS_PER_BLOCK:
        return _unpacked_kernel(x, w_t_pad, b_pad, y)
    steps = batch // _ROWS_PER_BLOCK
    cr = _ROWS_PER_BLOCK // _GROUPS
    yp = (y.reshape(steps, _GROUPS, cr).swapaxes(1, 2)
          .reshape(steps * cr, _GROUPS).astype(jnp.float32))

    body = functools.partial(_packed_ce_kernel, steps=steps,
                             inv_b=1.0 / batch, d=d)
    cost = pl.CostEstimate(
        flops=2 * batch * d * _GW + 8 * batch * _GW,
        transcendentals=batch * _GW + batch,
        bytes_accessed=batch * d * 4 + d * _LANES * 4 + batch * 4,
    )
    out = pl.pallas_call(
        body,
        out_shape=jax.ShapeDtypeStruct((8, _LANES), jnp.float32),
        grid=(steps,),
        in_specs=[
            pl.BlockSpec((_ROWS_PER_BLOCK, d), lambda i: (i, 0)),
            pl.BlockSpec((d, _LANES), lambda i: (0, 0)),
            pl.BlockSpec((1, _LANES), lambda i: (0, 0)),
            pl.BlockSpec((cr, _GROUPS), lambda i: (i, 0)),
        ],
        out_specs=pl.BlockSpec((8, _LANES), lambda i: (0, 0)),
        scratch_shapes=[
            pltpu.VMEM((8, _LANES), jnp.float32),
            pltpu.VMEM((_GROUPS * d, _LANES), jnp.float32),
            pltpu.VMEM((1, _LANES), jnp.float32),
            pltpu.VMEM((_LANES, _LANES), jnp.float32),
            pltpu.VMEM((_GROUPS, _LANES), jnp.float32),
        ],
        compiler_params=pltpu.CompilerParams(
            dimension_semantics=("arbitrary",),
            vmem_limit_bytes=_VMEM_LIMIT,
        ),
        cost_estimate=cost,
    )(x, w_t_pad, b_pad, yp)
    return out[0, 0]


def _unpacked_ce_kernel(x_ref, wt_ref, b_ref, y_ref, out_ref, acc_ref,
                        *, steps, inv_b, total_rows):
    i = pl.program_id(0)
    tm = x_ref.shape[0]

    @pl.when(i == 0)
    def _init():
        acc_ref[...] = jnp.zeros_like(acc_ref)

    logits = jnp.dot(x_ref[...], wt_ref[...],
                     preferred_element_type=jnp.float32)
    logits = logits + b_ref[...]
    m = jnp.max(logits, axis=-1, keepdims=True)
    lse = m + jnp.log(jnp.sum(jnp.exp(logits - m), axis=-1, keepdims=True))
    col = jax.lax.broadcasted_iota(jnp.int32, logits.shape, 1)
    picked = jnp.sum(jnp.where(col == y_ref[...], logits, 0.0),
                     axis=-1, keepdims=True)
    loss = lse - picked
    row = jax.lax.broadcasted_iota(jnp.int32, (tm, 1), 0) + i * tm
    acc_ref[...] += jnp.where(row < total_rows, loss, 0.0)

    @pl.when(i == steps - 1)
    def _finalize():
        out_ref[...] = jnp.broadcast_to(
            jnp.sum(acc_ref[...]) * inv_b, out_ref.shape)


def _unpacked_kernel(x, w_t_pad, b_pad, y):
    batch, d = x.shape
    tm = min(2048, max(8, -(-batch // 8) * 8))
    steps = pl.cdiv(batch, tm)
    y2 = y.reshape(batch, 1).astype(jnp.int32)
    body = functools.partial(_unpacked_ce_kernel, steps=steps,
                             inv_b=1.0 / batch, total_rows=batch)
    out = pl.pallas_call(
        body,
        out_shape=jax.ShapeDtypeStruct((8, _LANES), jnp.float32),
        grid=(steps,),
        in_specs=[
            pl.BlockSpec((tm, d), lambda i: (i, 0)),
            pl.BlockSpec((d, _LANES), lambda i: (0, 0)),
            pl.BlockSpec((1, _LANES), lambda i: (0, 0)),
            pl.BlockSpec((tm, 1), lambda i: (i, 0)),
        ],
        out_specs=pl.BlockSpec((8, _LANES), lambda i: (0, 0)),
        scratch_shapes=[pltpu.VMEM((tm, 1), jnp.float32)],
        compiler_params=pltpu.CompilerParams(
            dimension_semantics=("arbitrary",),
            vmem_limit_bytes=_VMEM_LIMIT,
        ),
    )(x, w_t_pad, b_pad, y2)
    return out[0, 0]
